# Initial kernel scaffold; baseline (speedup 1.0000x reference)
import jax
import jax.numpy as jnp
from jax.experimental import pallas as pl


def kernel(features, edge_index, edge_vals, W1, b1, W2, b2):
    raise NotImplementedError("write your pallas kernel here")


# speedup vs baseline: 4.7439x; 4.7439x over previous
import functools

import jax
import jax.numpy as jnp
from jax import lax
from jax.experimental import pallas as pl
from jax.experimental.pallas import tpu as pltpu
from jax.experimental.pallas import tpu_sc as plsc

N = 10000
E = 320000
D = 128
NC = 2
NS = 16
NW = NC * NS
CH = 128
NCHUNKS = E // CH
BASE = NCHUNKS // NW
EXTRA = NCHUNKS - BASE * NW
ZR = 208
RPS = 624
TAIL = N - NS * RPS


def _sc_spmm(features, src, dst, vals):
    mesh = plsc.VectorSubcoreMesh(core_axis_name="c", subcore_axis_name="s")

    @functools.partial(
        pl.kernel,
        out_type=jax.ShapeDtypeStruct((NC, N, D), jnp.float32),
        mesh=mesh,
        scratch_types=[
            pltpu.VMEM((CH,), jnp.int32),
            pltpu.VMEM((CH,), jnp.int32),
            pltpu.VMEM((CH,), jnp.float32),
            pltpu.VMEM((CH, D), jnp.float32),
            pltpu.VMEM((ZR, D), jnp.float32),
            pltpu.VMEM_SHARED((N, D), jnp.float32),
            pltpu.SemaphoreType.DMA,
        ],
        compiler_params=pltpu.CompilerParams(needs_layout_passes=False),
    )
    def k(feat_hbm, src_hbm, dst_hbm, vals_hbm, out_hbm,
          src_v, dst_v, vals_v, rows_v, zbuf, acc, sem):
        c = lax.axis_index("c")
        s = lax.axis_index("s")
        wid = s * NC + c

        def zero_row(r, _):
            for d in range(D // 16):
                zbuf[r, pl.ds(d * 16, 16)] = jnp.zeros((16,), jnp.float32)
            return _
        lax.fori_loop(0, ZR, zero_row, None)
        for j in range(RPS // ZR):
            pltpu.sync_copy(zbuf, acc.at[pl.ds(s * RPS + j * ZR, ZR)])

        @pl.when(s == 0)
        def _():
            pltpu.sync_copy(zbuf.at[pl.ds(0, TAIL)],
                            acc.at[pl.ds(NS * RPS, TAIL)])
        plsc.subcore_barrier()

        def do_chunk(g):
            off = g * CH
            pltpu.sync_copy(src_hbm.at[pl.ds(off, CH)], src_v)
            pltpu.sync_copy(dst_hbm.at[pl.ds(off, CH)], dst_v)
            pltpu.sync_copy(vals_hbm.at[pl.ds(off, CH)], vals_v)
            pltpu.async_copy(feat_hbm.at[src_v], rows_v, sem).wait()

            def scale_edge(e, _):
                vv = plsc.load_gather(vals_v, [jnp.full((16,), e, jnp.int32)])
                for d in range(D // 16):
                    sl = pl.ds(d * 16, 16)
                    rows_v[e, sl] = rows_v[e, sl] * vv
                return _
            lax.fori_loop(0, CH, scale_edge, None)
            pltpu.sync_copy(rows_v, acc.at[dst_v], add=True)

        def chunk_body(i, _):
            do_chunk(i * NW + wid)
            return _
        lax.fori_loop(0, BASE, chunk_body, None)

        @pl.when(wid < EXTRA)
        def _():
            do_chunk(BASE * NW + wid)

        plsc.subcore_barrier()
        for j in range(RPS // ZR):
            off = s * RPS + j * ZR
            pltpu.sync_copy(acc.at[pl.ds(off, ZR)],
                            out_hbm.at[c].at[pl.ds(off, ZR)])

        @pl.when(s == 0)
        def _():
            pltpu.sync_copy(acc.at[pl.ds(NS * RPS, TAIL)],
                            out_hbm.at[c].at[pl.ds(NS * RPS, TAIL)])

    return k(features, src, dst, vals)


def _tc_combine(features, partial, W1, b1, W2, b2):
    BN = 1000
    bias = (b1 + b2).reshape(1, D)
    p0 = partial[0]
    p1 = partial[1]

    def body(f_ref, p0_ref, p1_ref, w1_ref, w2_ref, b_ref, o_ref):
        lap = p0_ref[...] + p1_ref[...]
        f = f_ref[...]
        m1 = lap + f
        m2 = lap * f
        dn = (((1,), (1,)), ((), ()))
        o_ref[...] = (
            lax.dot_general(m1, w1_ref[...], dn,
                            preferred_element_type=jnp.float32)
            + lax.dot_general(m2, w2_ref[...], dn,
                              preferred_element_type=jnp.float32)
            + b_ref[...]
        )

    row_spec = pl.BlockSpec((BN, D), lambda i: (i, 0))
    full_spec = pl.BlockSpec((D, D), lambda i: (0, 0))
    return pl.pallas_call(
        body,
        grid=(N // BN,),
        in_specs=[row_spec, row_spec, row_spec, full_spec, full_spec,
                  pl.BlockSpec((1, D), lambda i: (0, 0))],
        out_specs=row_spec,
        out_shape=jax.ShapeDtypeStruct((N, D), jnp.float32),
    )(features, p0, p1, W1, W2, bias)


@jax.jit
def kernel(features, edge_index, edge_vals, W1, b1, W2, b2):
    dst = edge_index[0]
    src = edge_index[1]
    partial = _sc_spmm(features, src, dst, edge_vals)
    return _tc_combine(features, partial, W1, b1, W2, b2)
